# baseline (speedup 1.0000x reference)
import jax
import jax.numpy as jnp
from jax import lax
from jax.experimental import pallas as pl
from jax.experimental.pallas import tpu as pltpu
from jax.experimental.pallas import tpu_sc as plsc

S = 2048
SPLIT = 512
S_BACK = S - SPLIT
D_MODEL = 1024
D_FF = 2048
E = 8
D_FF_E = 512

T_FFN = 256
T_R = 256
NTR = S_BACK // T_R
TG = 128
NT = 32
P = NT * TG
NW = 32
CHUNK = S_BACK // NW
RG = P // NW
GC = 32


def _top2(x, w, d):
    l = jnp.dot(x, w, preferred_element_type=jnp.float32) + d
    iota_e = lax.broadcasted_iota(jnp.int32, l.shape, 1)
    m1 = jnp.max(l, axis=1, keepdims=True)
    a1 = jnp.min(jnp.where(l == m1, iota_e, E), axis=1, keepdims=True)
    masked = jnp.where(iota_e == a1, -jnp.inf, l)
    m2 = jnp.max(masked, axis=1, keepdims=True)
    a2 = jnp.min(jnp.where(masked == m2, iota_e, E), axis=1, keepdims=True)
    g1v = 1.0 / (1.0 + jnp.exp(m1 - m2))
    oh0 = (iota_e == a1).astype(jnp.float32)
    oh1 = (iota_e == a2).astype(jnp.float32)
    return oh0, oh1, 1.0 - g1v, g1v


def _router_body(x_ref, w_ref, d_ref, g0_ref, g1_ref, p0_ref, p1_ref,
                 te_ref, t0_s, t1_s, off_s, oi_s, c0_s, c1_s):
    p = pl.program_id(0)
    t = pl.program_id(1)
    oh0, oh1, g0v, g1v = _top2(x_ref[...], w_ref[...], d_ref[...])

    g0_ref[...] = jnp.broadcast_to(g0v, (T_R, 16))
    g1_ref[...] = jnp.broadcast_to(g1v, (T_R, 16))

    @pl.when((p == 0) & (t == 0))
    def _():
        for e in range(E):
            t0_s[e] = 0
            t1_s[e] = 0

    @pl.when(p == 0)
    def _():
        p0_ref[...] = jnp.zeros((T_R,), jnp.int32)
        p1_ref[...] = jnp.zeros((T_R,), jnp.int32)
        te_ref[...] = jnp.zeros((NT,), jnp.int32)
        for e in range(E):
            t0_s[e] = t0_s[e] + jnp.sum(oh0[:, e]).astype(jnp.int32)
            t1_s[e] = t1_s[e] + jnp.sum(oh1[:, e]).astype(jnp.int32)

    @pl.when((p == 1) & (t == 0))
    def _():
        acc = 0
        for e in range(E):
            tot = t0_s[e] + t1_s[e]
            pt = ((tot + (TG - 1)) >> 7) << 7
            off_s[e] = acc
            acc = acc + pt
            oi_s[e] = acc
            c0_s[e] = 0
            c1_s[e] = t0_s[e]

    @pl.when(p == 1)
    def _():
        iota_t = lax.broadcasted_iota(jnp.int32, (NT,), 0) * TG
        tev = jnp.zeros((NT,), jnp.int32)
        for e in range(E):
            tev = tev + (iota_t >= oi_s[e]).astype(jnp.int32)
        te_ref[...] = jnp.minimum(tev, E - 1)
        rows = lax.broadcasted_iota(jnp.int32, (T_R, T_R), 0)
        cols = lax.broadcasted_iota(jnp.int32, (T_R, T_R), 1)
        tri = (rows >= cols).astype(jnp.float32)
        pre0 = jnp.dot(tri, oh0, preferred_element_type=jnp.float32)
        pre1 = jnp.dot(tri, oh1, preferred_element_type=jnp.float32)
        iota_e = lax.broadcasted_iota(jnp.int32, (T_R, E), 1)
        base0 = jnp.zeros((T_R, E), jnp.int32)
        base1 = jnp.zeros((T_R, E), jnp.int32)
        for e in range(E):
            base0 = jnp.where(iota_e == e, off_s[e] + c0_s[e], base0)
            base1 = jnp.where(iota_e == e, off_s[e] + c1_s[e], base1)
        pos0 = jnp.sum(oh0 * (base0 + pre0 - 1).astype(jnp.float32), axis=1)
        pos1 = jnp.sum(oh1 * (base1 + pre1 - 1).astype(jnp.float32), axis=1)
        p0_ref[...] = pos0.astype(jnp.int32)
        p1_ref[...] = pos1.astype(jnp.int32)
        for e in range(E):
            c0_s[e] = c0_s[e] + jnp.sum(oh0[:, e]).astype(jnp.int32)
            c1_s[e] = c1_s[e] + jnp.sum(oh1[:, e]).astype(jnp.int32)


def _ffn_body(x_ref, w1_ref, b1_ref, w2_ref, b2_ref, y_ref):
    h = jnp.dot(x_ref[...], w1_ref[...], preferred_element_type=jnp.float32)
    h = jax.nn.gelu(h + b1_ref[...])
    y = jnp.dot(h, w2_ref[...], preferred_element_type=jnp.float32)
    y_ref[...] = y + b2_ref[...]


def _grouped_body(te_ref, xs_ref, we1_ref, be1_ref, we2_ref, be2_ref,
                  out_ref):
    del te_ref
    h = jnp.dot(xs_ref[...], we1_ref[0], preferred_element_type=jnp.float32)
    h = jax.nn.gelu(h + be1_ref[0])
    o = jnp.dot(h, we2_ref[0], preferred_element_type=jnp.float32)
    out_ref[...] = o + be2_ref[0]


def _routing_body(p0_hbm, p1_hbm, x_hbm, xs_hbm, i0v, i1v, rows, sem):
    w = lax.axis_index("s") * 2 + lax.axis_index("c")
    base_t = w * CHUNK
    pltpu.sync_copy(p0_hbm.at[pl.ds(base_t, CHUNK)], i0v)
    pltpu.sync_copy(p1_hbm.at[pl.ds(base_t, CHUNK)], i1v)
    pltpu.sync_copy(x_hbm.at[pl.ds(base_t, CHUNK)], rows)
    pltpu.async_copy(rows, xs_hbm.at[i0v], sem).wait()
    pltpu.async_copy(rows, xs_hbm.at[i1v], sem).wait()


def _combine_body(eo_hbm, yb_hbm, p0_hbm, p1_hbm, g0_hbm, g1_hbm, out_hbm,
                  i0v, i1v, g0m, g1m, b0, b1, by, ob, sem0, sem1):
    w = lax.axis_index("s") * 2 + lax.axis_index("c")
    base_t = w * CHUNK
    for c in range(CHUNK // 16):
        tb = base_t + c * 16
        pltpu.sync_copy(p0_hbm.at[pl.ds(tb, 16)], i0v)
        pltpu.sync_copy(p1_hbm.at[pl.ds(tb, 16)], i1v)
        pltpu.sync_copy(g0_hbm.at[pl.ds(tb, 16)], g0m)
        pltpu.sync_copy(g1_hbm.at[pl.ds(tb, 16)], g1m)
        cp0 = pltpu.async_copy(eo_hbm.at[i0v], b0, sem0)
        cp1 = pltpu.async_copy(eo_hbm.at[i1v], b1, sem1)
        pltpu.sync_copy(yb_hbm.at[pl.ds(tb, 16)], by)
        cp0.wait()
        cp1.wait()
        for t in range(16):
            ga = g0m[t]
            gb = g1m[t]

            def col_body(j, _, t=t, ga=ga, gb=gb):
                sl = pl.ds(j * 16, 16)
                ob[t, sl] = by[t, sl] + ga * b0[t, sl] + gb * b1[t, sl]
                return _
            lax.fori_loop(0, D_MODEL // 16, col_body, 0)
        pltpu.sync_copy(ob, out_hbm.at[pl.ds(tb, 16)])


def kernel(x, id, weight, delta, W1, b1, W2, b2, We1, be1, We2, be2):
    del id
    f32 = jnp.float32
    i32 = jnp.int32
    xf = x.reshape(S, D_MODEL)
    x_back = xf[SPLIT:]

    g0, g1, pos0, pos1, tile_e = pl.pallas_call(
        _router_body,
        grid=(2, NTR),
        in_specs=[
            pl.BlockSpec((T_R, D_MODEL), lambda p, t: (t, 0)),
            pl.BlockSpec((D_MODEL, E), lambda p, t: (0, 0)),
            pl.BlockSpec((1, E), lambda p, t: (0, 0)),
        ],
        out_specs=[
            pl.BlockSpec((T_R, 16), lambda p, t: (t, 0)),
            pl.BlockSpec((T_R, 16), lambda p, t: (t, 0)),
            pl.BlockSpec((T_R,), lambda p, t: (t,)),
            pl.BlockSpec((T_R,), lambda p, t: (t,)),
            pl.BlockSpec((NT,), lambda p, t: (0,)),
        ],
        out_shape=[
            jax.ShapeDtypeStruct((S_BACK, 16), f32),
            jax.ShapeDtypeStruct((S_BACK, 16), f32),
            jax.ShapeDtypeStruct((S_BACK,), i32),
            jax.ShapeDtypeStruct((S_BACK,), i32),
            jax.ShapeDtypeStruct((NT,), i32),
        ],
        scratch_shapes=[pltpu.SMEM((E,), i32)] * 6,
    )(x_back, weight, delta.reshape(1, E))

    y = pl.pallas_call(
        _ffn_body,
        grid=(S // T_FFN,),
        in_specs=[
            pl.BlockSpec((T_FFN, D_MODEL), lambda t: (t, 0)),
            pl.BlockSpec((D_MODEL, D_FF), lambda t: (0, 0)),
            pl.BlockSpec((1, D_FF), lambda t: (0, 0)),
            pl.BlockSpec((D_FF, D_MODEL), lambda t: (0, 0)),
            pl.BlockSpec((1, D_MODEL), lambda t: (0, 0)),
        ],
        out_specs=pl.BlockSpec((T_FFN, D_MODEL), lambda t: (t, 0)),
        out_shape=jax.ShapeDtypeStruct((S, D_MODEL), f32),
    )(xf, W1, b1.reshape(1, D_FF), W2, b2.reshape(1, D_MODEL))

    mesh = plsc.VectorSubcoreMesh(core_axis_name="c", subcore_axis_name="s")
    xs = pl.kernel(
        _routing_body,
        out_type=jax.ShapeDtypeStruct((P, D_MODEL), f32),
        mesh=mesh,
        scratch_types=[
            pltpu.VMEM((CHUNK,), i32),
            pltpu.VMEM((CHUNK,), i32),
            pltpu.VMEM((CHUNK, D_MODEL), f32),
            pltpu.SemaphoreType.DMA,
        ],
    )(pos0, pos1, x_back)

    eo = pl.pallas_call(
        _grouped_body,
        grid_spec=pltpu.PrefetchScalarGridSpec(
            num_scalar_prefetch=1,
            grid=(NT,),
            in_specs=[
                pl.BlockSpec((TG, D_MODEL), lambda t, te: (t, 0)),
                pl.BlockSpec((1, D_MODEL, D_FF_E), lambda t, te: (te[t], 0, 0)),
                pl.BlockSpec((1, 1, D_FF_E), lambda t, te: (te[t], 0, 0)),
                pl.BlockSpec((1, D_FF_E, D_MODEL), lambda t, te: (te[t], 0, 0)),
                pl.BlockSpec((1, 1, D_MODEL), lambda t, te: (te[t], 0, 0)),
            ],
            out_specs=pl.BlockSpec((TG, D_MODEL), lambda t, te: (t, 0)),
        ),
        out_shape=jax.ShapeDtypeStruct((P, D_MODEL), f32),
    )(tile_e, xs, We1, be1.reshape(E, 1, D_FF_E), We2,
      be2.reshape(E, 1, D_MODEL))

    out_back = pl.kernel(
        _combine_body,
        out_type=jax.ShapeDtypeStruct((S_BACK, D_MODEL), f32),
        mesh=mesh,
        scratch_types=[
            pltpu.VMEM((16,), i32),
            pltpu.VMEM((16,), i32),
            pltpu.VMEM((16, 16), f32),
            pltpu.VMEM((16, 16), f32),
            pltpu.VMEM((16, D_MODEL), f32),
            pltpu.VMEM((16, D_MODEL), f32),
            pltpu.VMEM((16, D_MODEL), f32),
            pltpu.VMEM((16, D_MODEL), f32),
            pltpu.SemaphoreType.DMA,
            pltpu.SemaphoreType.DMA,
        ],
    )(eo, y[SPLIT:], pos0, pos1, g0, g1)

    out = jnp.concatenate([y[:SPLIT], out_back], axis=0)
    return out.reshape(1, S, D_MODEL)
